# baseline (speedup 1.0000x reference)
import functools

import jax
import jax.numpy as jnp
from jax import lax
from jax.experimental import pallas as pl
from jax.experimental.pallas import tpu as pltpu
from jax.experimental.pallas import tpu_sc as plsc

B = 32
C = 384
H = 56
W = 56
CH = 4
RING = 4
LEAD = 2
RA = RING - LEAD
NCHUNK = C // CH
NGROUP = NCHUNK // RING
NSUB = 16


HB = B // 2
CHALF = C
NC_HALF = (C // 2) // CH


def _permute_half(x, idx_i32, batch0):
    mesh = plsc.VectorSubcoreMesh(core_axis_name="c", subcore_axis_name="s")
    num_cores = mesh.num_cores

    @functools.partial(
        pl.kernel,
        out_type=jax.ShapeDtypeStruct((HB, C, H, W), jnp.float32),
        mesh=mesh,
        scratch_types=[
            pltpu.VMEM((C + 16,), jnp.int32),
            pltpu.VMEM_SHARED((NSUB, RING, CH, H, W), jnp.float32),
            pltpu.SemaphoreType.DMA((RING,)),
            pltpu.SemaphoreType.DMA((RING,)),
        ],
    )
    def k(in_hbm, idx_hbm, out_hbm, idx_v, sbuf, gsem, psem):
        cid = lax.axis_index("c")
        sid = lax.axis_index("s")
        wid = sid * num_cores + cid
        bat = wid // 2
        c0 = (wid % 2) * (C // 2)
        buf = sbuf.at[sid]
        pltpu.sync_copy(idx_hbm, idx_v.at[pl.ds(0, C)])

        def gather_chunk(i, slot):
            v = idx_v[pl.ds(c0 + i * CH, 16)]
            for j in range(CH):
                pltpu.async_copy(
                    in_hbm.at[batch0 + bat, v[j]], buf.at[slot, j],
                    gsem.at[slot],
                )

        def wait_gather(slot):
            for _j in range(CH):
                pltpu.make_async_copy(
                    in_hbm.at[0, 0], buf.at[slot, 0], gsem.at[slot]
                ).wait()

        def put(i, slot):
            pltpu.async_copy(
                buf.at[slot], out_hbm.at[bat, pl.ds(c0 + i * CH, CH)],
                psem.at[slot],
            )

        def wait_put(slot):
            pltpu.make_async_copy(
                buf.at[slot], out_hbm.at[0, pl.ds(0, CH)], psem.at[slot]
            ).wait()

        for j in range(RA):
            gather_chunk(j, j)

        def step(i, need_wait_put):
            s = i % RING
            sg = (i + RA) % RING
            if need_wait_put:
                wait_put(sg)

            @pl.when(i + RA < NC_HALF)
            def _():
                gather_chunk(i + RA, sg)

            wait_gather(s)
            put(i, s)

        for i in range(RING):
            step(i, i >= LEAD)

        def body(q, _):
            i0 = q * RING
            for s in range(RING):
                step(i0 + s, True)
            return 0

        lax.fori_loop(1, NC_HALF // RING, body, 0)
        for k in range(LEAD):
            wait_put((NC_HALF - LEAD + k) % RING)

    return k(x, idx_i32)


def _permute_tc(x, idx_i32, batch0, nb):
    def body(idx_ref, in_ref, out_ref):
        out_ref[...] = in_ref[...]

    return pl.pallas_call(
        body,
        grid_spec=pltpu.PrefetchScalarGridSpec(
            num_scalar_prefetch=1,
            grid=(nb, C),
            in_specs=[
                pl.BlockSpec(
                    (1, 1, H, W),
                    lambda b, c, idx: (batch0 + b, idx[c], 0, 0),
                )
            ],
            out_specs=pl.BlockSpec(
                (1, 1, H, W), lambda b, c, idx: (b, c, 0, 0)
            ),
        ),
        out_shape=jax.ShapeDtypeStruct((nb, C, H, W), jnp.float32),
        compiler_params=pltpu.CompilerParams(
            dimension_semantics=("parallel", "arbitrary")
        ),
    )(idx_i32, x)


def kernel(input, indices, indices_inverse):
    idx = indices.astype(jnp.int32)
    a = _permute_half(input, idx, 0)
    b = _permute_tc(input, idx, HB, B - HB)
    return jnp.concatenate([a, b], axis=0)


# speedup vs baseline: 3.8947x; 3.8947x over previous
import functools

import jax
import jax.numpy as jnp
from jax import lax
from jax.experimental import pallas as pl
from jax.experimental.pallas import tpu as pltpu
from jax.experimental.pallas import tpu_sc as plsc

B = 32
C = 384
H = 56
W = 56
CH = 4
RING = 4
LEAD = 2
RA = RING - LEAD
NCHUNK = C // CH
NGROUP = NCHUNK // RING
NSUB = 16


HB = B // 2
CHALF = C
NC_HALF = (C // 2) // CH


def _permute_half(x, idx_i32, batch0):
    mesh = plsc.VectorSubcoreMesh(core_axis_name="c", subcore_axis_name="s")
    num_cores = mesh.num_cores

    @functools.partial(
        pl.kernel,
        out_type=jax.ShapeDtypeStruct((HB, C, H, W), jnp.float32),
        mesh=mesh,
        scratch_types=[
            pltpu.VMEM((C + 16,), jnp.int32),
            pltpu.VMEM_SHARED((NSUB, RING, CH, H, W), jnp.float32),
            pltpu.SemaphoreType.DMA((RING,)),
            pltpu.SemaphoreType.DMA((RING,)),
        ],
    )
    def k(in_hbm, idx_hbm, out_hbm, idx_v, sbuf, gsem, psem):
        cid = lax.axis_index("c")
        sid = lax.axis_index("s")
        wid = sid * num_cores + cid
        bat = wid // 2
        c0 = (wid % 2) * (C // 2)
        buf = sbuf.at[sid]
        pltpu.sync_copy(idx_hbm, idx_v.at[pl.ds(0, C)])

        def gather_chunk(i, slot):
            v = idx_v[pl.ds(c0 + i * CH, 16)]
            for j in range(CH):
                pltpu.async_copy(
                    in_hbm.at[batch0 + bat, v[j]], buf.at[slot, j],
                    gsem.at[slot],
                )

        def wait_gather(slot):
            for _j in range(CH):
                pltpu.make_async_copy(
                    in_hbm.at[0, 0], buf.at[slot, 0], gsem.at[slot]
                ).wait()

        def put(i, slot):
            pltpu.async_copy(
                buf.at[slot], out_hbm.at[bat, pl.ds(c0 + i * CH, CH)],
                psem.at[slot],
            )

        def wait_put(slot):
            pltpu.make_async_copy(
                buf.at[slot], out_hbm.at[0, pl.ds(0, CH)], psem.at[slot]
            ).wait()

        for j in range(RA):
            gather_chunk(j, j)

        def step(i, need_wait_put):
            s = i % RING
            sg = (i + RA) % RING
            if need_wait_put:
                wait_put(sg)

            @pl.when(i + RA < NC_HALF)
            def _():
                gather_chunk(i + RA, sg)

            wait_gather(s)
            put(i, s)

        for i in range(RING):
            step(i, i >= LEAD)

        def body(q, _):
            i0 = q * RING
            for s in range(RING):
                step(i0 + s, True)
            return 0

        lax.fori_loop(1, NC_HALF // RING, body, 0)
        for k in range(LEAD):
            wait_put((NC_HALF - LEAD + k) % RING)

    return k(x, idx_i32)


def _permute_tc(x, idx_i32, batch0, nb):
    def body(idx_ref, in_ref, out_ref):
        out_ref[...] = in_ref[...]

    return pl.pallas_call(
        body,
        grid_spec=pltpu.PrefetchScalarGridSpec(
            num_scalar_prefetch=1,
            grid=(C,),
            in_specs=[
                pl.BlockSpec(
                    (nb, 1, H, W),
                    lambda c, idx: (batch0 // nb, idx[c], 0, 0),
                )
            ],
            out_specs=pl.BlockSpec(
                (nb, 1, H, W), lambda c, idx: (0, c, 0, 0)
            ),
        ),
        out_shape=jax.ShapeDtypeStruct((nb, C, H, W), jnp.float32),
        compiler_params=pltpu.CompilerParams(
            dimension_semantics=("arbitrary",)
        ),
    )(idx_i32, x)


def kernel(input, indices, indices_inverse):
    idx = indices.astype(jnp.int32)
    a = _permute_half(input, idx, 0)
    b = _permute_tc(input, idx, HB, B - HB)
    return jnp.concatenate([a, b], axis=0)


# speedup vs baseline: 3.9411x; 1.0119x over previous
import functools

import jax
import jax.numpy as jnp
from jax import lax
from jax.experimental import pallas as pl
from jax.experimental.pallas import tpu as pltpu
from jax.experimental.pallas import tpu_sc as plsc

B = 32
C = 384
H = 56
W = 56
CH = 4
RING = 4
LEAD = 2
RA = RING - LEAD
NSUB = 16

B_SC = 24
QPW = 3
CQ = C // 4
NCQ = CQ // CH


def _permute_sc(x, idx_i32):
    mesh = plsc.VectorSubcoreMesh(core_axis_name="c", subcore_axis_name="s")
    num_cores = mesh.num_cores

    @functools.partial(
        pl.kernel,
        out_type=jax.ShapeDtypeStruct((B_SC, C, H, W), jnp.float32),
        mesh=mesh,
        scratch_types=[
            pltpu.VMEM((C + 16,), jnp.int32),
            pltpu.VMEM_SHARED((NSUB, RING, CH, H, W), jnp.float32),
            pltpu.SemaphoreType.DMA((RING,)),
            pltpu.SemaphoreType.DMA((RING,)),
        ],
    )
    def k(in_hbm, idx_hbm, out_hbm, idx_v, sbuf, gsem, psem):
        cid = lax.axis_index("c")
        sid = lax.axis_index("s")
        wid = sid * num_cores + cid
        buf = sbuf.at[sid]
        pltpu.sync_copy(idx_hbm, idx_v.at[pl.ds(0, C)])

        def run_quarter(bat, c0):
            def gather_chunk(i, slot):
                v = idx_v[pl.ds(c0 + i * CH, 16)]
                for j in range(CH):
                    pltpu.async_copy(
                        in_hbm.at[bat, v[j]], buf.at[slot, j], gsem.at[slot]
                    )

            def wait_gather(slot):
                for _j in range(CH):
                    pltpu.make_async_copy(
                        in_hbm.at[0, 0], buf.at[slot, 0], gsem.at[slot]
                    ).wait()

            def put(i, slot):
                pltpu.async_copy(
                    buf.at[slot],
                    out_hbm.at[bat, pl.ds(c0 + i * CH, CH)],
                    psem.at[slot],
                )

            def wait_put(slot):
                pltpu.make_async_copy(
                    buf.at[slot], out_hbm.at[0, pl.ds(0, CH)], psem.at[slot]
                ).wait()

            for j in range(RA):
                gather_chunk(j, j)

            def step(i, need_wait_put):
                s = i % RING
                sg = (i + RA) % RING
                if need_wait_put:
                    wait_put(sg)

                @pl.when(i + RA < NCQ)
                def _():
                    gather_chunk(i + RA, sg)

                wait_gather(s)
                put(i, s)

            for i in range(RING):
                step(i, i >= LEAD)

            def body(q, _):
                i0 = q * RING
                for s in range(RING):
                    step(i0 + s, True)
                return 0

            lax.fori_loop(1, NCQ // RING, body, 0)
            for kk in range(LEAD):
                wait_put((NCQ - LEAD + kk) % RING)

        for t in range(QPW):
            q = wid * QPW + t
            run_quarter(q // 4, (q % 4) * CQ)

    return k(x, idx_i32)


def _permute_tc(x, idx_i32, batch0, nb):
    def body(idx_ref, in_ref, out_ref):
        out_ref[...] = in_ref[...]

    return pl.pallas_call(
        body,
        grid_spec=pltpu.PrefetchScalarGridSpec(
            num_scalar_prefetch=1,
            grid=(C,),
            in_specs=[
                pl.BlockSpec(
                    (nb, 1, H, W),
                    lambda c, idx: (batch0 // nb, idx[c], 0, 0),
                )
            ],
            out_specs=pl.BlockSpec(
                (nb, 1, H, W), lambda c, idx: (0, c, 0, 0)
            ),
        ),
        out_shape=jax.ShapeDtypeStruct((nb, C, H, W), jnp.float32),
        compiler_params=pltpu.CompilerParams(
            dimension_semantics=("arbitrary",)
        ),
    )(idx_i32, x)


def kernel(input, indices, indices_inverse):
    idx = indices.astype(jnp.int32)
    a = _permute_sc(input, idx)
    b = _permute_tc(input, idx, B_SC, B - B_SC)
    return jnp.concatenate([a, b], axis=0)


# speedup vs baseline: 5.1223x; 1.2997x over previous
import functools

import jax
import jax.numpy as jnp
from jax import lax
from jax.experimental import pallas as pl
from jax.experimental.pallas import tpu as pltpu
from jax.experimental.pallas import tpu_sc as plsc

B = 32
C = 384
H = 56
W = 56
CH = 4
RING = 4
LEAD = 2
RA = RING - LEAD
NCHUNK = C // CH
NGROUP = NCHUNK // RING
NSUB = 16


def _permute(x, idx_i32):
    mesh = plsc.VectorSubcoreMesh(core_axis_name="c", subcore_axis_name="s")
    num_cores = mesh.num_cores

    @functools.partial(
        pl.kernel,
        out_type=jax.ShapeDtypeStruct((B, C, H, W), jnp.float32),
        mesh=mesh,
        scratch_types=[
            pltpu.VMEM((C + 16,), jnp.int32),
            pltpu.VMEM_SHARED((NSUB, RING, CH, H, W), jnp.float32),
            pltpu.SemaphoreType.DMA((RING,)),
            pltpu.SemaphoreType.DMA((RING,)),
        ],
    )
    def k(in_hbm, idx_hbm, out_hbm, idx_v, sbuf, gsem, psem):
        cid = lax.axis_index("c")
        sid = lax.axis_index("s")
        wid = sid * num_cores + cid
        buf = sbuf.at[sid]
        pltpu.sync_copy(idx_hbm, idx_v.at[pl.ds(0, C)])

        def gather_chunk(i, slot):
            v = idx_v[pl.ds(i * CH, 16)]
            for j in range(CH):
                pltpu.async_copy(
                    in_hbm.at[wid, v[j]], buf.at[slot, j], gsem.at[slot]
                )

        def wait_gather(slot):
            for _j in range(CH):
                pltpu.make_async_copy(
                    in_hbm.at[0, 0], buf.at[slot, 0], gsem.at[slot]
                ).wait()

        def put(i, slot):
            pltpu.async_copy(
                buf.at[slot], out_hbm.at[wid, pl.ds(i * CH, CH)], psem.at[slot]
            )

        def wait_put(slot):
            pltpu.make_async_copy(
                buf.at[slot], out_hbm.at[0, pl.ds(0, CH)], psem.at[slot]
            ).wait()

        for j in range(RA):
            gather_chunk(j, j)

        def step(i, need_wait_put):
            s = i % RING
            sg = (i + RA) % RING
            if need_wait_put:
                wait_put(sg)

            @pl.when(i + RA < NCHUNK)
            def _():
                gather_chunk(i + RA, sg)

            wait_gather(s)
            put(i, s)

        for i in range(RING):
            step(i, i >= LEAD)

        def body(q, _):
            i0 = q * RING
            for s in range(RING):
                step(i0 + s, True)
            return 0

        lax.fori_loop(1, NGROUP, body, 0)
        for k in range(LEAD):
            wait_put((NCHUNK - LEAD + k) % RING)

    return k(x, idx_i32)


def kernel(input, indices, indices_inverse):
    idx = indices.astype(jnp.int32)
    return _permute(input, idx)
